# baseline (speedup 1.0000x reference)
import functools

import jax
import jax.numpy as jnp
from jax import lax
from jax.experimental import pallas as pl
from jax.experimental.pallas import tpu as pltpu
from jax.experimental.pallas import tpu_sc as plsc

N_ROWS = 100000
N_CATES = 1000
D = 128
B = 4096
HIST = 20
MC = 4
HPAD = 1024

NC, NS = 2, 16
NW = NC * NS
BPW = B // NW
STRIPE = N_ROWS // NW
CHB = 8
NCH = BPW // CHB
F32 = jnp.float32
I32 = jnp.int32


def _iota16():
    return lax.broadcasted_iota(I32, (16,), 0)


def _gp_body(user_w, item_w, cates_t, clens_t, users, items, ihm, ihl, uhm, uhl,
             su_o, pi_o, pu_o, si_o, hu_o, hi_o,
             users_v, items_v, ihl_v, uhl_v, ihm_v, uhm_v,
             rowsA_v, rowsB_v, acc_v, stat_v, ccat_v, clh_v, icat_v, icl_v,
             h16_v, cidx_v, icidx_v, sem, semA, semB, semC):
    wid = lax.axis_index("c") * NS + lax.axis_index("s")
    b0 = wid * BPW
    iota = _iota16()

    pltpu.sync_copy(users.at[pl.ds(b0, BPW)], users_v)
    pltpu.sync_copy(items.at[pl.ds(b0, BPW)], items_v)
    pltpu.sync_copy(ihl.at[pl.ds(b0, BPW)], ihl_v.at[pl.ds(0, BPW)])
    pltpu.sync_copy(uhl.at[pl.ds(b0, BPW)], uhl_v.at[pl.ds(0, BPW)])
    pltpu.sync_copy(ihm.at[pl.ds(b0 * HIST, BPW * HIST)], ihm_v)
    pltpu.sync_copy(uhm.at[pl.ds(b0 * HIST, BPW * HIST)], uhm_v)

    def cidx_h(g, _):
        ids = ihm_v[pl.ds(g * 16, 16)]
        for c in range(MC):
            plsc.store_scatter(cidx_v, [iota * MC + (g * 16 * MC + c)],
                               ids * MC + c)
        return 0
    lax.fori_loop(0, BPW * HIST // 16, cidx_h, 0)

    def cidx_i(g, _):
        ids = items_v[pl.ds(g * 16, 16)]
        for c in range(MC):
            plsc.store_scatter(icidx_v, [iota * MC + (g * 16 * MC + c)],
                               ids * MC + c)
        return 0
    lax.fori_loop(0, BPW // 16, cidx_i, 0)

    pltpu.async_copy(cates_t.at[cidx_v], ccat_v, semC)
    pltpu.async_copy(clens_t.at[ihm_v], clh_v, semC)
    pltpu.async_copy(cates_t.at[icidx_v], icat_v, semC)
    pltpu.async_copy(clens_t.at[items_v], icl_v, semC)

    pltpu.async_copy(user_w.at[users_v], stat_v, sem).wait()
    pltpu.sync_copy(stat_v, su_o.at[pl.ds(b0, BPW)])
    pltpu.async_copy(item_w.at[items_v], stat_v, sem).wait()
    pltpu.sync_copy(stat_v, si_o.at[pl.ds(b0, BPW)])

    def pool(hidx_v, len_v, table, out):
        def gsrc(c):
            return table.at[hidx_v.at[pl.ds(c * (CHB * HIST), CHB * HIST)]]

        def compute(c, rows):
            def b_body(bb, _):
                b = c * CHB + bb
                lnv16 = len_v[pl.ds(b, 16)]
                ln = lnv16[0]
                inv16 = 1.0 / (lnv16.astype(F32) + 1e-9)
                invs = jnp.broadcast_to(inv16[0], (16,))
                zeros = jnp.zeros((16,), F32)
                accs = [jnp.zeros((16,), F32) for _ in range(8)]
                for l in range(HIST):
                    w = jnp.where(l < ln, invs, zeros)
                    base = bb * HIST + l
                    for v in range(8):
                        accs[v] = accs[v] + rows[base, pl.ds(v * 16, 16)] * w
                for v in range(8):
                    acc_v[b, pl.ds(v * 16, 16)] = accs[v]
                return 0

            lax.fori_loop(0, CHB, b_body, 0)

        pltpu.async_copy(gsrc(0), rowsA_v, semA)

        def pair(p, _):
            c0 = 2 * p
            pltpu.make_async_copy(gsrc(c0), rowsA_v, semA).wait()
            pltpu.async_copy(gsrc(c0 + 1), rowsB_v, semB)
            compute(c0, rowsA_v)
            pltpu.make_async_copy(gsrc(c0 + 1), rowsB_v, semB).wait()

            @pl.when(p < NCH // 2 - 1)
            def _():
                pltpu.async_copy(gsrc(c0 + 2), rowsA_v, semA)

            compute(c0 + 1, rowsB_v)
            return 0

        lax.fori_loop(0, NCH // 2, pair, 0)
        pltpu.sync_copy(acc_v, out.at[pl.ds(b0, BPW)])

    pool(ihm_v, ihl_v, item_w, pi_o)
    pool(uhm_v, uhl_v, user_w, pu_o)

    pltpu.make_async_copy(cates_t.at[cidx_v], ccat_v, semC).wait()
    pltpu.make_async_copy(clens_t.at[ihm_v], clh_v, semC).wait()
    pltpu.make_async_copy(cates_t.at[icidx_v], icat_v, semC).wait()
    pltpu.make_async_copy(clens_t.at[items_v], icl_v, semC).wait()

    def zero_h16(_unused):
        def z(i, _):
            h16_v[pl.ds(i * 16, 16)] = jnp.zeros((16,), F32)
            return 0
        lax.fori_loop(0, 16 * HPAD // 16, z, 0)

    def hu_chunk(cb, _):
        zero_h16(None)
        lnv = ihl_v[pl.ds(cb * 16, 16)]
        lnf = lnv.astype(F32) + 1e-9

        def l_body(l, _):
            posv = iota * HIST + (cb * 16 * HIST + l)
            clv = plsc.load_gather(clh_v, [posv])
            wb = 1.0 / ((clv.astype(F32) + 1e-9) * lnf)
            vl = l < lnv
            for c in range(MC):
                cid = plsc.load_gather(ccat_v, [posv * MC + c])
                val = vl & (c < clv)
                cidc = jnp.where(val, cid, 0)
                plsc.addupdate_scatter(h16_v, [iota * HPAD + cidc], wb, mask=val)
            return 0

        lax.fori_loop(0, HIST, l_body, 0)
        pltpu.sync_copy(h16_v, hu_o.at[pl.ds((b0 + cb * 16) * HPAD, 16 * HPAD)])
        return 0

    lax.fori_loop(0, BPW // 16, hu_chunk, 0)

    def hi_chunk(cb, _):
        zero_h16(None)
        posv = iota + cb * 16
        clv = icl_v[pl.ds(cb * 16, 16)]
        wb = 1.0 / (clv.astype(F32) + 1e-9)
        for c in range(MC):
            cid = plsc.load_gather(icat_v, [posv * MC + c])
            val = c < clv
            cidc = jnp.where(val, cid, 0)
            plsc.addupdate_scatter(h16_v, [iota * HPAD + cidc], wb, mask=val)
        pltpu.sync_copy(h16_v, hi_o.at[pl.ds((b0 + cb * 16) * HPAD, 16 * HPAD)])
        return 0

    lax.fori_loop(0, BPW // 16, hi_chunk, 0)


def _gather_pool(user_w, item_w, cates_t, clens_t, users, items, ihm_f, ihl, uhm_f, uhl):
    mesh = plsc.VectorSubcoreMesh(core_axis_name="c", subcore_axis_name="s")
    f = functools.partial(
        pl.kernel,
        out_type=[
            jax.ShapeDtypeStruct((B, D), F32),
            jax.ShapeDtypeStruct((B, D), F32),
            jax.ShapeDtypeStruct((B, D), F32),
            jax.ShapeDtypeStruct((B, D), F32),
            jax.ShapeDtypeStruct((B * HPAD,), F32),
            jax.ShapeDtypeStruct((B * HPAD,), F32),
        ],
        mesh=mesh,
        scratch_types=[
            pltpu.VMEM((BPW,), I32),
            pltpu.VMEM((BPW,), I32),
            pltpu.VMEM((BPW + 16,), I32),
            pltpu.VMEM((BPW + 16,), I32),
            pltpu.VMEM((BPW * HIST,), I32),
            pltpu.VMEM((BPW * HIST,), I32),
            pltpu.VMEM((CHB * HIST, D), F32),
            pltpu.VMEM((CHB * HIST, D), F32),
            pltpu.VMEM((BPW, D), F32),
            pltpu.VMEM((BPW, D), F32),
            pltpu.VMEM((BPW * HIST * MC,), I32),
            pltpu.VMEM((BPW * HIST,), I32),
            pltpu.VMEM((BPW * MC,), I32),
            pltpu.VMEM((BPW,), I32),
            pltpu.VMEM((16 * HPAD,), F32),
            pltpu.VMEM((BPW * HIST * MC,), I32),
            pltpu.VMEM((BPW * MC,), I32),
            pltpu.SemaphoreType.DMA,
            pltpu.SemaphoreType.DMA,
            pltpu.SemaphoreType.DMA,
            pltpu.SemaphoreType.DMA,
        ],
        compiler_params=pltpu.CompilerParams(needs_layout_passes=False),
    )(_gp_body)
    return f(user_w, item_w, cates_t, clens_t, users, items, ihm_f, ihl, uhm_f, uhl)


LCAP = 3200
STRIPE_W = 3128
LAST_W = N_ROWS - (NW - 1) * STRIPE_W


def _sc_body(buf, items, pu, out,
             items_v, table_v, dst_v, src_v, dstc_v, srcc_v, rows_v,
             cp0_v, cp1_v, sem, semi0, semi1, semo0, semo1):
    wid = lax.axis_index("c") * NS + lax.axis_index("s")
    r0 = wid * STRIPE_W
    rlim = jnp.minimum(jnp.int32(STRIPE_W), jnp.int32(N_ROWS) - r0)
    iota = _iota16()

    nfull = rlim >> 7
    rem8 = (rlim & 127) >> 3

    def cp2(c2, _):
        a0 = r0 + (c2 * 2) * 128
        a1 = a0 + 128
        pltpu.async_copy(buf.at[pl.ds(a0, 128)], cp0_v, semi0)
        pltpu.async_copy(buf.at[pl.ds(a1, 128)], cp1_v, semi1)
        pltpu.make_async_copy(buf.at[pl.ds(a0, 128)], cp0_v, semi0).wait()
        pltpu.async_copy(cp0_v, out.at[pl.ds(a0, 128)], semo0)
        pltpu.make_async_copy(buf.at[pl.ds(a1, 128)], cp1_v, semi1).wait()
        pltpu.async_copy(cp1_v, out.at[pl.ds(a1, 128)], semo1)
        pltpu.make_async_copy(cp0_v, out.at[pl.ds(a0, 128)], semo0).wait()
        pltpu.make_async_copy(cp1_v, out.at[pl.ds(a1, 128)], semo1).wait()
        return 0
    lax.fori_loop(0, nfull >> 1, cp2, 0)

    @pl.when((nfull & 1) == 1)
    def _():
        a0 = r0 + (nfull - 1) * 128
        pltpu.sync_copy(buf.at[pl.ds(a0, 128)], cp0_v)
        pltpu.sync_copy(cp0_v, out.at[pl.ds(a0, 128)])

    def cp8(k, _):
        a0 = r0 + nfull * 128 + k * 8
        pltpu.sync_copy(buf.at[pl.ds(a0, 8)], cp0_v.at[pl.ds(0, 8)])
        pltpu.sync_copy(cp0_v.at[pl.ds(0, 8)], out.at[pl.ds(a0, 8)])
        return 0
    lax.fori_loop(0, rem8, cp8, 0)

    pltpu.sync_copy(items, items_v)

    def tinit(t, _):
        table_v[pl.ds(t * 16, 16)] = jnp.full((16,), -1, I32)
        return 0
    lax.fori_loop(0, (STRIPE_W + 15) // 16, tinit, 0)

    def smax_pass(g, changed):
        ids = items_v[pl.ds(g * 16, 16)]
        bvec = iota + g * 16
        lid = ids - r0
        inm = (lid >= 0) & (lid < rlim)
        lidc = jnp.where(inm, lid, 0)
        rb = plsc.load_gather(table_v, [lidc], mask=inm)
        rb = jnp.where(inm, rb, bvec)
        fix = inm & (bvec > rb)
        plsc.store_scatter(table_v, [lidc], bvec, mask=fix)
        return changed + plsc.all_reduce_population_count(fix)

    def wcond(ch):
        return ch > 0

    def wbody(_):
        chv = lax.fori_loop(0, B // 16, smax_pass, jnp.zeros((16,), I32))
        return jnp.max(chv)

    lax.while_loop(wcond, wbody, jnp.int32(1))

    def comp(t, cntv):
        lidv = iota + t * 16
        rb = table_v[pl.ds(t * 16, 16)]
        m = (rb >= 0) & (lidv < rlim)
        mi = jnp.where(m, 1, 0).astype(I32)
        pos = cntv + plsc.cumsum(mi) - 1
        posc = jnp.where(m, pos, 0)
        plsc.store_scatter(dst_v, [posc], lidv + r0, mask=m)
        plsc.store_scatter(src_v, [posc], rb, mask=m)
        return cntv + plsc.all_reduce_population_count(m)

    cntv = lax.fori_loop(0, (STRIPE_W + 15) // 16, comp,
                         jnp.zeros((16,), I32))
    cnt = jnp.max(cntv)
    nch = (cnt + 63) >> 6

    @pl.when(cnt > 0)
    def _():
        w0d = dst_v[pl.ds(0, 16)][0]
        w0s = src_v[pl.ds(0, 16)][0]

        def padg(t, _):
            posp = iota + t * 16
            m = (posp >= cnt) & (posp < nch * 64)
            plsc.store_scatter(dst_v, [jnp.where(m, posp, 0)],
                               jnp.full((16,), 1, I32) * w0d, mask=m)
            plsc.store_scatter(src_v, [jnp.where(m, posp, 0)],
                               jnp.full((16,), 1, I32) * w0s, mask=m)
            return 0
        lax.fori_loop(cnt >> 4, jnp.minimum(nch * 4, LCAP // 16), padg, 0)

        def sc_chunk(j, _):
            for k in range(4):
                dstc_v[pl.ds(k * 16, 16)] = dst_v[pl.ds(j * 64 + k * 16, 16)]
                srcc_v[pl.ds(k * 16, 16)] = src_v[pl.ds(j * 64 + k * 16, 16)]
            pltpu.async_copy(pu.at[srcc_v], rows_v, sem).wait()
            pltpu.async_copy(rows_v, out.at[dstc_v], sem).wait()
            return 0
        lax.fori_loop(0, nch, sc_chunk, 0)


def _scatter_update(buf, items, pu):
    mesh = plsc.VectorSubcoreMesh(core_axis_name="c", subcore_axis_name="s")
    f = functools.partial(
        pl.kernel,
        out_type=[jax.ShapeDtypeStruct((N_ROWS, D), F32)],
        mesh=mesh,
        scratch_types=[
            pltpu.VMEM((B,), I32),
            pltpu.VMEM((STRIPE_W + 8,), I32),
            pltpu.VMEM((LCAP,), I32),
            pltpu.VMEM((LCAP,), I32),
            pltpu.VMEM((64,), I32),
            pltpu.VMEM((64,), I32),
            pltpu.VMEM((64, D), F32),
            pltpu.VMEM((128, D), F32),
            pltpu.VMEM((128, D), F32),
            pltpu.SemaphoreType.DMA,
            pltpu.SemaphoreType.DMA,
            pltpu.SemaphoreType.DMA,
            pltpu.SemaphoreType.DMA,
            pltpu.SemaphoreType.DMA,
        ],
        compiler_params=pltpu.CompilerParams(needs_layout_passes=False),
    )(_sc_body)
    (nb,) = f(buf, items, pu)
    return nb


BM = 512


def _prenorm(x, g, b, W1, b1, W2, b2):
    m = jnp.mean(x, axis=-1, keepdims=True)
    xc = x - m
    v = jnp.mean(xc * xc, axis=-1, keepdims=True)
    h = xc * lax.rsqrt(v + 1e-5) * g + b
    h = jnp.maximum(jnp.dot(h, W1, preferred_element_type=F32) + b1, 0.0)
    h = jnp.dot(h, W2, preferred_element_type=F32) + b2
    return h + x


def _l2n(x):
    n = jnp.sqrt(jnp.sum(x * x, axis=-1, keepdims=True))
    return x / jnp.maximum(n, 1e-12)


def _mlp_body(su, pi, pu, si, hu, hi, wcat,
              p1, p2, p3, p4, ue_o, ie_o):
    avgu = jnp.dot(hu[...], wcat[...], preferred_element_type=F32)
    avgi = jnp.dot(hi[...], wcat[...], preferred_element_type=F32)
    uf = jnp.concatenate([su[...], pi[...], avgu], axis=1)
    itf = jnp.concatenate([si[...], avgi, pu[...]], axis=1)

    def blk(p, x):
        return _prenorm(x, p['ln_g'][...], p['ln_b'][...], p['W1'][...],
                        p['b1'][...], p['W2'][...], p['b2'][...])

    ue = blk(p1, uf) + blk(p2, uf)
    ie = blk(p3, itf) + blk(p4, itf)
    ue_o[...] = _l2n(ue)
    ie_o[...] = _l2n(ie)


def _mlp(su, pi, pu, si, hu, hi, wcat, p1, p2, p3, p4):
    bspec_d = pl.BlockSpec((BM, D), lambda i: (i, 0))
    bspec_h = pl.BlockSpec((BM, HPAD), lambda i: (i, 0))
    full = lambda a: pl.BlockSpec(a.shape, lambda i: tuple(0 for _ in a.shape))
    pspec = lambda p: {k: full(v) for k, v in p.items()}
    return pl.pallas_call(
        _mlp_body,
        grid=(B // BM,),
        in_specs=[bspec_d, bspec_d, bspec_d, bspec_d, bspec_h, bspec_h,
                  full(wcat), pspec(p1), pspec(p2), pspec(p3), pspec(p4)],
        out_specs=[pl.BlockSpec((BM, 3 * D), lambda i: (i, 0))] * 2,
        out_shape=[jax.ShapeDtypeStruct((B, 3 * D), F32)] * 2,
    )(su, pi, pu, si, hu, hi, wcat, p1, p2, p3, p4)


def kernel(params, users, items, item_history_matrix, item_history_len,
           user_history_matrix, user_history_len, cates, cate_lens):
    users = users.astype(I32)
    items = items.astype(I32)
    ihm_f = item_history_matrix.astype(I32).reshape(B * HIST)
    uhm_f = user_history_matrix.astype(I32).reshape(B * HIST)
    ihl = item_history_len.astype(I32)
    uhl = user_history_len.astype(I32)
    cates_t = cates.astype(I32).reshape(N_ROWS * MC)
    clens_t = cate_lens.astype(I32)

    su, pi, pu, si, huf, hif = _gather_pool(
        params['user_emb_w'], params['item_emb_w'], cates_t, clens_t,
        users, items, ihm_f, ihl, uhm_f, uhl)

    hu = huf.reshape(B, HPAD)
    hi = hif.reshape(B, HPAD)
    wcat = jnp.pad(params['cate_emb_w'], ((0, HPAD - N_CATES), (0, 0)))

    new_buffer = _scatter_update(params['user_history_buffer'], items, pu)

    ue, ie = _mlp(su, pi, pu, si, hu, hi, wcat,
                  params['user_mlp'], params['user_mlp_2'],
                  params['item_mlp'], params['item_mlp_2'])
    return ue, ie, new_buffer


# speedup vs baseline: 1.0250x; 1.0250x over previous
import functools

import jax
import jax.numpy as jnp
from jax import lax
from jax.experimental import pallas as pl
from jax.experimental.pallas import tpu as pltpu
from jax.experimental.pallas import tpu_sc as plsc

N_ROWS = 100000
N_CATES = 1000
D = 128
B = 4096
HIST = 20
MC = 4
HPAD = 1024

NC, NS = 2, 16
NW = NC * NS
BPW = B // NW
STRIPE = N_ROWS // NW
CHB = 8
NCH = BPW // CHB
F32 = jnp.float32
I32 = jnp.int32


def _iota16():
    return lax.broadcasted_iota(I32, (16,), 0)


def _gp_body(user_w, item_w, cates_t, clens_t, users, items, ihm, ihl, uhm, uhl,
             su_o, pi_o, pu_o, si_o, hu_o, hi_o,
             users_v, items_v, ihl_v, uhl_v, ihm_v, uhm_v,
             rowsA_v, rowsB_v, acc_v, ccat_v, clh_v, icat_v, icl_v,
             h16a_v, h16b_v, cidx_v, icidx_v, sem, semA, semB, semC,
             semH0, semH1):
    wid = lax.axis_index("c") * NS + lax.axis_index("s")
    b0 = wid * BPW
    iota = _iota16()

    pltpu.sync_copy(users.at[pl.ds(b0, BPW)], users_v)
    pltpu.sync_copy(items.at[pl.ds(b0, BPW)], items_v)
    pltpu.sync_copy(ihl.at[pl.ds(b0, BPW)], ihl_v.at[pl.ds(0, BPW)])
    pltpu.sync_copy(uhl.at[pl.ds(b0, BPW)], uhl_v.at[pl.ds(0, BPW)])
    pltpu.sync_copy(ihm.at[pl.ds(b0 * HIST, BPW * HIST)], ihm_v)
    pltpu.sync_copy(uhm.at[pl.ds(b0 * HIST, BPW * HIST)], uhm_v)

    def cidx_h(g, _):
        ids = ihm_v[pl.ds(g * 16, 16)]
        for c in range(MC):
            plsc.store_scatter(cidx_v, [iota * MC + (g * 16 * MC + c)],
                               ids * MC + c)
        return 0
    lax.fori_loop(0, BPW * HIST // 16, cidx_h, 0)

    def cidx_i(g, _):
        ids = items_v[pl.ds(g * 16, 16)]
        for c in range(MC):
            plsc.store_scatter(icidx_v, [iota * MC + (g * 16 * MC + c)],
                               ids * MC + c)
        return 0
    lax.fori_loop(0, BPW // 16, cidx_i, 0)

    pltpu.async_copy(cates_t.at[cidx_v], ccat_v, semC)
    pltpu.async_copy(clens_t.at[ihm_v], clh_v, semC)
    pltpu.async_copy(cates_t.at[icidx_v], icat_v, semC)
    pltpu.async_copy(clens_t.at[items_v], icl_v, semC)

    pltpu.async_copy(user_w.at[users_v], acc_v, sem).wait()
    pltpu.sync_copy(acc_v, su_o.at[pl.ds(b0, BPW)])
    pltpu.async_copy(item_w.at[items_v], acc_v, sem).wait()
    pltpu.sync_copy(acc_v, si_o.at[pl.ds(b0, BPW)])

    def pool(hidx_v, len_v, table, out):
        def gsrc(c):
            return table.at[hidx_v.at[pl.ds(c * (CHB * HIST), CHB * HIST)]]

        def compute(c, rows):
            def b_body(bb, _):
                b = c * CHB + bb
                lnv16 = len_v[pl.ds(b, 16)]
                ln = lnv16[0]
                inv16 = 1.0 / (lnv16.astype(F32) + 1e-9)
                invs = jnp.broadcast_to(inv16[0], (16,))
                zeros = jnp.zeros((16,), F32)
                accs = [jnp.zeros((16,), F32) for _ in range(8)]
                for l in range(HIST):
                    w = jnp.where(l < ln, invs, zeros)
                    base = bb * HIST + l
                    for v in range(8):
                        accs[v] = accs[v] + rows[base, pl.ds(v * 16, 16)] * w
                for v in range(8):
                    acc_v[b, pl.ds(v * 16, 16)] = accs[v]
                return 0

            lax.fori_loop(0, CHB, b_body, 0)

        pltpu.async_copy(gsrc(0), rowsA_v, semA)

        def pair(p, _):
            c0 = 2 * p
            pltpu.make_async_copy(gsrc(c0), rowsA_v, semA).wait()
            pltpu.async_copy(gsrc(c0 + 1), rowsB_v, semB)
            compute(c0, rowsA_v)
            pltpu.make_async_copy(gsrc(c0 + 1), rowsB_v, semB).wait()

            @pl.when(p < NCH // 2 - 1)
            def _():
                pltpu.async_copy(gsrc(c0 + 2), rowsA_v, semA)

            compute(c0 + 1, rowsB_v)
            return 0

        lax.fori_loop(0, NCH // 2, pair, 0)
        pltpu.sync_copy(acc_v, out.at[pl.ds(b0, BPW)])

    pool(ihm_v, ihl_v, item_w, pi_o)
    pool(uhm_v, uhl_v, user_w, pu_o)

    pltpu.make_async_copy(cates_t.at[cidx_v], ccat_v, semC).wait()
    pltpu.make_async_copy(clens_t.at[ihm_v], clh_v, semC).wait()
    pltpu.make_async_copy(cates_t.at[icidx_v], icat_v, semC).wait()
    pltpu.make_async_copy(clens_t.at[items_v], icl_v, semC).wait()

    def zero16(buf):
        def z(i, _):
            buf[pl.ds(i * 16, 16)] = jnp.zeros((16,), F32)
            return 0
        lax.fori_loop(0, 16 * HPAD // 16, z, 0)

    def fill_user(cb, buf):
        lnv = ihl_v[pl.ds(cb * 16, 16)]
        lnf = lnv.astype(F32) + 1e-9

        def l_body(l, _):
            posv = iota * HIST + (cb * 16 * HIST + l)
            clv = plsc.load_gather(clh_v, [posv])
            wb = 1.0 / ((clv.astype(F32) + 1e-9) * lnf)
            vl = l < lnv
            for c in range(MC):
                cid = plsc.load_gather(ccat_v, [posv * MC + c])
                val = vl & (c < clv)
                cidc = jnp.where(val, cid, 0)
                plsc.addupdate_scatter(buf, [iota * HPAD + cidc], wb, mask=val)
            return 0

        lax.fori_loop(0, HIST, l_body, 0)

    def fill_item(cb, buf):
        posv = iota + cb * 16
        clv = icl_v[pl.ds(cb * 16, 16)]
        wb = 1.0 / (clv.astype(F32) + 1e-9)
        for c in range(MC):
            cid = plsc.load_gather(icat_v, [posv * MC + c])
            val = c < clv
            cidc = jnp.where(val, cid, 0)
            plsc.addupdate_scatter(buf, [iota * HPAD + cidc], wb, mask=val)

    def run_hist(fill, out_o):
        def pair(p, _):
            for buf, semx, dcb in ((h16a_v, semH0, 0), (h16b_v, semH1, 1)):
                cb = 2 * p + dcb

                @pl.when(p > 0)
                def _():
                    pltpu.make_async_copy(
                        buf,
                        out_o.at[pl.ds((b0 + (cb - 2) * 16) * HPAD, 16 * HPAD)],
                        semx).wait()

                zero16(buf)
                fill(cb, buf)
                pltpu.async_copy(
                    buf, out_o.at[pl.ds((b0 + cb * 16) * HPAD, 16 * HPAD)],
                    semx)
            return 0

        lax.fori_loop(0, 4, pair, 0)
        pltpu.make_async_copy(
            h16a_v, out_o.at[pl.ds((b0 + 6 * 16) * HPAD, 16 * HPAD)],
            semH0).wait()
        pltpu.make_async_copy(
            h16b_v, out_o.at[pl.ds((b0 + 7 * 16) * HPAD, 16 * HPAD)],
            semH1).wait()

    run_hist(fill_user, hu_o)
    run_hist(fill_item, hi_o)


def _gather_pool(user_w, item_w, cates_t, clens_t, users, items, ihm_f, ihl, uhm_f, uhl):
    mesh = plsc.VectorSubcoreMesh(core_axis_name="c", subcore_axis_name="s")
    f = functools.partial(
        pl.kernel,
        out_type=[
            jax.ShapeDtypeStruct((B, D), F32),
            jax.ShapeDtypeStruct((B, D), F32),
            jax.ShapeDtypeStruct((B, D), F32),
            jax.ShapeDtypeStruct((B, D), F32),
            jax.ShapeDtypeStruct((B * HPAD,), F32),
            jax.ShapeDtypeStruct((B * HPAD,), F32),
        ],
        mesh=mesh,
        scratch_types=[
            pltpu.VMEM((BPW,), I32),
            pltpu.VMEM((BPW,), I32),
            pltpu.VMEM((BPW + 16,), I32),
            pltpu.VMEM((BPW + 16,), I32),
            pltpu.VMEM((BPW * HIST,), I32),
            pltpu.VMEM((BPW * HIST,), I32),
            pltpu.VMEM((CHB * HIST, D), F32),
            pltpu.VMEM((CHB * HIST, D), F32),
            pltpu.VMEM((BPW, D), F32),
            pltpu.VMEM((BPW * HIST * MC,), I32),
            pltpu.VMEM((BPW * HIST,), I32),
            pltpu.VMEM((BPW * MC,), I32),
            pltpu.VMEM((BPW,), I32),
            pltpu.VMEM((16 * HPAD,), F32),
            pltpu.VMEM((16 * HPAD,), F32),
            pltpu.VMEM((BPW * HIST * MC,), I32),
            pltpu.VMEM((BPW * MC,), I32),
            pltpu.SemaphoreType.DMA,
            pltpu.SemaphoreType.DMA,
            pltpu.SemaphoreType.DMA,
            pltpu.SemaphoreType.DMA,
            pltpu.SemaphoreType.DMA,
            pltpu.SemaphoreType.DMA,
        ],
        compiler_params=pltpu.CompilerParams(needs_layout_passes=False),
    )(_gp_body)
    return f(user_w, item_w, cates_t, clens_t, users, items, ihm_f, ihl, uhm_f, uhl)


LCAP = 3200
STRIPE_W = 3128
LAST_W = N_ROWS - (NW - 1) * STRIPE_W


def _sc_body(buf, items, pu, out,
             items_v, table_v, dst_v, src_v, dstc_v, srcc_v, rows_v,
             cp0_v, cp1_v, sem, semi0, semi1, semo0, semo1):
    wid = lax.axis_index("c") * NS + lax.axis_index("s")
    r0 = wid * STRIPE_W
    rlim = jnp.minimum(jnp.int32(STRIPE_W), jnp.int32(N_ROWS) - r0)
    iota = _iota16()

    nfull = rlim >> 7
    rem8 = (rlim & 127) >> 3

    def cp2(c2, _):
        a0 = r0 + (c2 * 2) * 128
        a1 = a0 + 128
        pltpu.async_copy(buf.at[pl.ds(a0, 128)], cp0_v, semi0)
        pltpu.async_copy(buf.at[pl.ds(a1, 128)], cp1_v, semi1)
        pltpu.make_async_copy(buf.at[pl.ds(a0, 128)], cp0_v, semi0).wait()
        pltpu.async_copy(cp0_v, out.at[pl.ds(a0, 128)], semo0)
        pltpu.make_async_copy(buf.at[pl.ds(a1, 128)], cp1_v, semi1).wait()
        pltpu.async_copy(cp1_v, out.at[pl.ds(a1, 128)], semo1)
        pltpu.make_async_copy(cp0_v, out.at[pl.ds(a0, 128)], semo0).wait()
        pltpu.make_async_copy(cp1_v, out.at[pl.ds(a1, 128)], semo1).wait()
        return 0
    lax.fori_loop(0, nfull >> 1, cp2, 0)

    @pl.when((nfull & 1) == 1)
    def _():
        a0 = r0 + (nfull - 1) * 128
        pltpu.sync_copy(buf.at[pl.ds(a0, 128)], cp0_v)
        pltpu.sync_copy(cp0_v, out.at[pl.ds(a0, 128)])

    def cp8(k, _):
        a0 = r0 + nfull * 128 + k * 8
        pltpu.sync_copy(buf.at[pl.ds(a0, 8)], cp0_v.at[pl.ds(0, 8)])
        pltpu.sync_copy(cp0_v.at[pl.ds(0, 8)], out.at[pl.ds(a0, 8)])
        return 0
    lax.fori_loop(0, rem8, cp8, 0)

    pltpu.sync_copy(items, items_v)

    def tinit(t, _):
        table_v[pl.ds(t * 16, 16)] = jnp.full((16,), -1, I32)
        return 0
    lax.fori_loop(0, (STRIPE_W + 15) // 16, tinit, 0)

    def smax_pass(g, changed):
        ids = items_v[pl.ds(g * 16, 16)]
        bvec = iota + g * 16
        lid = ids - r0
        inm = (lid >= 0) & (lid < rlim)
        lidc = jnp.where(inm, lid, 0)
        rb = plsc.load_gather(table_v, [lidc], mask=inm)
        rb = jnp.where(inm, rb, bvec)
        fix = inm & (bvec > rb)
        plsc.store_scatter(table_v, [lidc], bvec, mask=fix)
        return changed + plsc.all_reduce_population_count(fix)

    def wcond(ch):
        return ch > 0

    def wbody(_):
        chv = lax.fori_loop(0, B // 16, smax_pass, jnp.zeros((16,), I32))
        return jnp.max(chv)

    lax.while_loop(wcond, wbody, jnp.int32(1))

    def comp(t, cntv):
        lidv = iota + t * 16
        rb = table_v[pl.ds(t * 16, 16)]
        m = (rb >= 0) & (lidv < rlim)
        mi = jnp.where(m, 1, 0).astype(I32)
        pos = cntv + plsc.cumsum(mi) - 1
        posc = jnp.where(m, pos, 0)
        plsc.store_scatter(dst_v, [posc], lidv + r0, mask=m)
        plsc.store_scatter(src_v, [posc], rb, mask=m)
        return cntv + plsc.all_reduce_population_count(m)

    cntv = lax.fori_loop(0, (STRIPE_W + 15) // 16, comp,
                         jnp.zeros((16,), I32))
    cnt = jnp.max(cntv)
    nch = (cnt + 63) >> 6

    @pl.when(cnt > 0)
    def _():
        w0d = dst_v[pl.ds(0, 16)][0]
        w0s = src_v[pl.ds(0, 16)][0]

        def padg(t, _):
            posp = iota + t * 16
            m = (posp >= cnt) & (posp < nch * 64)
            plsc.store_scatter(dst_v, [jnp.where(m, posp, 0)],
                               jnp.full((16,), 1, I32) * w0d, mask=m)
            plsc.store_scatter(src_v, [jnp.where(m, posp, 0)],
                               jnp.full((16,), 1, I32) * w0s, mask=m)
            return 0
        lax.fori_loop(cnt >> 4, jnp.minimum(nch * 4, LCAP // 16), padg, 0)

        def sc_chunk(j, _):
            for k in range(4):
                dstc_v[pl.ds(k * 16, 16)] = dst_v[pl.ds(j * 64 + k * 16, 16)]
                srcc_v[pl.ds(k * 16, 16)] = src_v[pl.ds(j * 64 + k * 16, 16)]
            pltpu.async_copy(pu.at[srcc_v], rows_v, sem).wait()
            pltpu.async_copy(rows_v, out.at[dstc_v], sem).wait()
            return 0
        lax.fori_loop(0, nch, sc_chunk, 0)


def _scatter_update(buf, items, pu):
    mesh = plsc.VectorSubcoreMesh(core_axis_name="c", subcore_axis_name="s")
    f = functools.partial(
        pl.kernel,
        out_type=[jax.ShapeDtypeStruct((N_ROWS, D), F32)],
        mesh=mesh,
        scratch_types=[
            pltpu.VMEM((B,), I32),
            pltpu.VMEM((STRIPE_W + 8,), I32),
            pltpu.VMEM((LCAP,), I32),
            pltpu.VMEM((LCAP,), I32),
            pltpu.VMEM((64,), I32),
            pltpu.VMEM((64,), I32),
            pltpu.VMEM((64, D), F32),
            pltpu.VMEM((128, D), F32),
            pltpu.VMEM((128, D), F32),
            pltpu.SemaphoreType.DMA,
            pltpu.SemaphoreType.DMA,
            pltpu.SemaphoreType.DMA,
            pltpu.SemaphoreType.DMA,
            pltpu.SemaphoreType.DMA,
        ],
        compiler_params=pltpu.CompilerParams(needs_layout_passes=False),
    )(_sc_body)
    (nb,) = f(buf, items, pu)
    return nb


BM = 512


def _prenorm(x, g, b, W1, b1, W2, b2):
    m = jnp.mean(x, axis=-1, keepdims=True)
    xc = x - m
    v = jnp.mean(xc * xc, axis=-1, keepdims=True)
    h = xc * lax.rsqrt(v + 1e-5) * g + b
    h = jnp.maximum(jnp.dot(h, W1, preferred_element_type=F32) + b1, 0.0)
    h = jnp.dot(h, W2, preferred_element_type=F32) + b2
    return h + x


def _l2n(x):
    n = jnp.sqrt(jnp.sum(x * x, axis=-1, keepdims=True))
    return x / jnp.maximum(n, 1e-12)


def _mlp_body(su, pi, pu, si, hu, hi, wcat,
              p1, p2, p3, p4, ue_o, ie_o):
    avgu = jnp.dot(hu[...], wcat[...], preferred_element_type=F32)
    avgi = jnp.dot(hi[...], wcat[...], preferred_element_type=F32)
    uf = jnp.concatenate([su[...], pi[...], avgu], axis=1)
    itf = jnp.concatenate([si[...], avgi, pu[...]], axis=1)

    def blk(p, x):
        return _prenorm(x, p['ln_g'][...], p['ln_b'][...], p['W1'][...],
                        p['b1'][...], p['W2'][...], p['b2'][...])

    ue = blk(p1, uf) + blk(p2, uf)
    ie = blk(p3, itf) + blk(p4, itf)
    ue_o[...] = _l2n(ue)
    ie_o[...] = _l2n(ie)


def _mlp(su, pi, pu, si, hu, hi, wcat, p1, p2, p3, p4):
    bspec_d = pl.BlockSpec((BM, D), lambda i: (i, 0))
    bspec_h = pl.BlockSpec((BM, HPAD), lambda i: (i, 0))
    full = lambda a: pl.BlockSpec(a.shape, lambda i: tuple(0 for _ in a.shape))
    pspec = lambda p: {k: full(v) for k, v in p.items()}
    return pl.pallas_call(
        _mlp_body,
        grid=(B // BM,),
        in_specs=[bspec_d, bspec_d, bspec_d, bspec_d, bspec_h, bspec_h,
                  full(wcat), pspec(p1), pspec(p2), pspec(p3), pspec(p4)],
        out_specs=[pl.BlockSpec((BM, 3 * D), lambda i: (i, 0))] * 2,
        out_shape=[jax.ShapeDtypeStruct((B, 3 * D), F32)] * 2,
    )(su, pi, pu, si, hu, hi, wcat, p1, p2, p3, p4)


def kernel(params, users, items, item_history_matrix, item_history_len,
           user_history_matrix, user_history_len, cates, cate_lens):
    users = users.astype(I32)
    items = items.astype(I32)
    ihm_f = item_history_matrix.astype(I32).reshape(B * HIST)
    uhm_f = user_history_matrix.astype(I32).reshape(B * HIST)
    ihl = item_history_len.astype(I32)
    uhl = user_history_len.astype(I32)
    cates_t = cates.astype(I32).reshape(N_ROWS * MC)
    clens_t = cate_lens.astype(I32)

    su, pi, pu, si, huf, hif = _gather_pool(
        params['user_emb_w'], params['item_emb_w'], cates_t, clens_t,
        users, items, ihm_f, ihl, uhm_f, uhl)

    hu = huf.reshape(B, HPAD)
    hi = hif.reshape(B, HPAD)
    wcat = jnp.pad(params['cate_emb_w'], ((0, HPAD - N_CATES), (0, 0)))

    new_buffer = _scatter_update(params['user_history_buffer'], items, pu)

    ue, ie = _mlp(su, pi, pu, si, hu, hi, wcat,
                  params['user_mlp'], params['user_mlp_2'],
                  params['item_mlp'], params['item_mlp_2'])
    return ue, ie, new_buffer


# speedup vs baseline: 1.2508x; 1.2203x over previous
import functools

import jax
import jax.numpy as jnp
from jax import lax
from jax.experimental import pallas as pl
from jax.experimental.pallas import tpu as pltpu
from jax.experimental.pallas import tpu_sc as plsc

N_ROWS = 100000
N_CATES = 1000
D = 128
B = 4096
HIST = 20
MC = 4
HPAD = 1024

NC, NS = 2, 16
NW = NC * NS
BPW = B // NW
STRIPE = N_ROWS // NW
CHB = 8
NCH = BPW // CHB
F32 = jnp.float32
I32 = jnp.int32


def _iota16():
    return lax.broadcasted_iota(I32, (16,), 0)


def _gp_body(user_w, item_w, cates_t, clens_t, users, items, ihm, ihl, uhm, uhl,
             su_o, pi_o, pu_o, si_o, hu_o, hi_o,
             users_v, items_v, ihl_v, uhl_v, ihm_v, uhm_v,
             rowsA_v, rowsB_v, acc_v, ccat_v, clh_v, icat_v, icl_v,
             h16a_v, h16b_v, cidx_v, icidx_v, sem, semA, semB, semC,
             semH0, semH1):
    wid = lax.axis_index("c") * NS + lax.axis_index("s")
    b0 = wid * BPW
    iota = _iota16()

    pltpu.sync_copy(users.at[pl.ds(b0, BPW)], users_v)
    pltpu.sync_copy(items.at[pl.ds(b0, BPW)], items_v)
    pltpu.sync_copy(ihl.at[pl.ds(b0, BPW)], ihl_v.at[pl.ds(0, BPW)])
    pltpu.sync_copy(uhl.at[pl.ds(b0, BPW)], uhl_v.at[pl.ds(0, BPW)])
    pltpu.sync_copy(ihm.at[pl.ds(b0 * HIST, BPW * HIST)], ihm_v)
    pltpu.sync_copy(uhm.at[pl.ds(b0 * HIST, BPW * HIST)], uhm_v)

    def cidx_h(g, _):
        ids = ihm_v[pl.ds(g * 16, 16)]
        for c in range(MC):
            plsc.store_scatter(cidx_v, [iota * MC + (g * 16 * MC + c)],
                               ids * MC + c)
        return 0
    lax.fori_loop(0, BPW * HIST // 16, cidx_h, 0)

    def cidx_i(g, _):
        ids = items_v[pl.ds(g * 16, 16)]
        for c in range(MC):
            plsc.store_scatter(icidx_v, [iota * MC + (g * 16 * MC + c)],
                               ids * MC + c)
        return 0
    lax.fori_loop(0, BPW // 16, cidx_i, 0)

    pltpu.async_copy(cates_t.at[cidx_v], ccat_v, semC)
    pltpu.async_copy(clens_t.at[ihm_v], clh_v, semC)
    pltpu.async_copy(cates_t.at[icidx_v], icat_v, semC)
    pltpu.async_copy(clens_t.at[items_v], icl_v, semC)

    pltpu.async_copy(user_w.at[users_v], acc_v, sem).wait()
    pltpu.sync_copy(acc_v, su_o.at[pl.ds(b0, BPW)])
    pltpu.async_copy(item_w.at[items_v], acc_v, sem).wait()
    pltpu.sync_copy(acc_v, si_o.at[pl.ds(b0, BPW)])

    def pool(hidx_v, len_v, table, out):
        def gsrc(c):
            return table.at[hidx_v.at[pl.ds(c * (CHB * HIST), CHB * HIST)]]

        def compute(c, rows):
            def b_body(bb, _):
                b = c * CHB + bb
                lnv16 = len_v[pl.ds(b, 16)]
                ln = lnv16[0]
                inv16 = 1.0 / (lnv16.astype(F32) + 1e-9)
                invs = jnp.broadcast_to(inv16[0], (16,))
                zeros = jnp.zeros((16,), F32)
                accs = [jnp.zeros((16,), F32) for _ in range(8)]
                for l in range(HIST):
                    w = jnp.where(l < ln, invs, zeros)
                    base = bb * HIST + l
                    for v in range(8):
                        accs[v] = accs[v] + rows[base, pl.ds(v * 16, 16)] * w
                for v in range(8):
                    acc_v[b, pl.ds(v * 16, 16)] = accs[v]
                return 0

            lax.fori_loop(0, CHB, b_body, 0)

        pltpu.async_copy(gsrc(0), rowsA_v, semA)

        def pair(p, _):
            c0 = 2 * p
            pltpu.make_async_copy(gsrc(c0), rowsA_v, semA).wait()
            pltpu.async_copy(gsrc(c0 + 1), rowsB_v, semB)
            compute(c0, rowsA_v)
            pltpu.make_async_copy(gsrc(c0 + 1), rowsB_v, semB).wait()

            @pl.when(p < NCH // 2 - 1)
            def _():
                pltpu.async_copy(gsrc(c0 + 2), rowsA_v, semA)

            compute(c0 + 1, rowsB_v)
            return 0

        lax.fori_loop(0, NCH // 2, pair, 0)
        pltpu.sync_copy(acc_v, out.at[pl.ds(b0, BPW)])

    pool(ihm_v, ihl_v, item_w, pi_o)
    pool(uhm_v, uhl_v, user_w, pu_o)

    pltpu.make_async_copy(cates_t.at[cidx_v], ccat_v, semC).wait()
    pltpu.make_async_copy(clens_t.at[ihm_v], clh_v, semC).wait()
    pltpu.make_async_copy(cates_t.at[icidx_v], icat_v, semC).wait()
    pltpu.make_async_copy(clens_t.at[items_v], icl_v, semC).wait()

    def zero16(buf):
        zeros = jnp.zeros((16,), F32)

        def z(i, _):
            for k in range(16):
                buf[pl.ds(i * 256 + k * 16, 16)] = zeros
            return 0
        lax.fori_loop(0, 16 * HPAD // 256, z, 0)

    def fill_user(cb, buf):
        lnv = ihl_v[pl.ds(cb * 16, 16)]
        lnf = lnv.astype(F32) + 1e-9

        def l_body(l, _):
            posv = iota * HIST + (cb * 16 * HIST + l)
            clv = plsc.load_gather(clh_v, [posv])
            wb = 1.0 / ((clv.astype(F32) + 1e-9) * lnf)
            vl = l < lnv
            for c in range(MC):
                cid = plsc.load_gather(ccat_v, [posv * MC + c])
                val = vl & (c < clv)
                cidc = jnp.where(val, cid, 0)
                plsc.addupdate_scatter(buf, [iota * HPAD + cidc], wb, mask=val)
            return 0

        lax.fori_loop(0, HIST, l_body, 0)

    def fill_item(cb, buf):
        posv = iota + cb * 16
        clv = icl_v[pl.ds(cb * 16, 16)]
        wb = 1.0 / (clv.astype(F32) + 1e-9)
        for c in range(MC):
            cid = plsc.load_gather(icat_v, [posv * MC + c])
            val = c < clv
            cidc = jnp.where(val, cid, 0)
            plsc.addupdate_scatter(buf, [iota * HPAD + cidc], wb, mask=val)

    def run_hist(fill, out_o):
        def pair(p, _):
            for buf, semx, dcb in ((h16a_v, semH0, 0), (h16b_v, semH1, 1)):
                cb = 2 * p + dcb

                @pl.when(p > 0)
                def _():
                    pltpu.make_async_copy(
                        buf,
                        out_o.at[pl.ds((b0 + (cb - 2) * 16) * HPAD, 16 * HPAD)],
                        semx).wait()

                zero16(buf)
                fill(cb, buf)
                pltpu.async_copy(
                    buf, out_o.at[pl.ds((b0 + cb * 16) * HPAD, 16 * HPAD)],
                    semx)
            return 0

        lax.fori_loop(0, 4, pair, 0)
        pltpu.make_async_copy(
            h16a_v, out_o.at[pl.ds((b0 + 6 * 16) * HPAD, 16 * HPAD)],
            semH0).wait()
        pltpu.make_async_copy(
            h16b_v, out_o.at[pl.ds((b0 + 7 * 16) * HPAD, 16 * HPAD)],
            semH1).wait()

    run_hist(fill_user, hu_o)
    run_hist(fill_item, hi_o)


def _gather_pool(user_w, item_w, cates_t, clens_t, users, items, ihm_f, ihl, uhm_f, uhl):
    mesh = plsc.VectorSubcoreMesh(core_axis_name="c", subcore_axis_name="s")
    f = functools.partial(
        pl.kernel,
        out_type=[
            jax.ShapeDtypeStruct((B, D), F32),
            jax.ShapeDtypeStruct((B, D), F32),
            jax.ShapeDtypeStruct((B, D), F32),
            jax.ShapeDtypeStruct((B, D), F32),
            jax.ShapeDtypeStruct((B * HPAD,), F32),
            jax.ShapeDtypeStruct((B * HPAD,), F32),
        ],
        mesh=mesh,
        scratch_types=[
            pltpu.VMEM((BPW,), I32),
            pltpu.VMEM((BPW,), I32),
            pltpu.VMEM((BPW + 16,), I32),
            pltpu.VMEM((BPW + 16,), I32),
            pltpu.VMEM((BPW * HIST,), I32),
            pltpu.VMEM((BPW * HIST,), I32),
            pltpu.VMEM((CHB * HIST, D), F32),
            pltpu.VMEM((CHB * HIST, D), F32),
            pltpu.VMEM((BPW, D), F32),
            pltpu.VMEM((BPW * HIST * MC,), I32),
            pltpu.VMEM((BPW * HIST,), I32),
            pltpu.VMEM((BPW * MC,), I32),
            pltpu.VMEM((BPW,), I32),
            pltpu.VMEM((16 * HPAD,), F32),
            pltpu.VMEM((16 * HPAD,), F32),
            pltpu.VMEM((BPW * HIST * MC,), I32),
            pltpu.VMEM((BPW * MC,), I32),
            pltpu.SemaphoreType.DMA,
            pltpu.SemaphoreType.DMA,
            pltpu.SemaphoreType.DMA,
            pltpu.SemaphoreType.DMA,
            pltpu.SemaphoreType.DMA,
            pltpu.SemaphoreType.DMA,
        ],
        compiler_params=pltpu.CompilerParams(needs_layout_passes=False),
    )(_gp_body)
    return f(user_w, item_w, cates_t, clens_t, users, items, ihm_f, ihl, uhm_f, uhl)


LCAP = 3200
STRIPE_W = 3128
LAST_W = N_ROWS - (NW - 1) * STRIPE_W


def _sc_body(buf, items, pu, out,
             items_v, table_v, dst_v, src_v, dstc_v, srcc_v, rows_v,
             cp0_v, cp1_v, sem, semi0, semi1, semo0, semo1):
    wid = lax.axis_index("c") * NS + lax.axis_index("s")
    r0 = wid * STRIPE_W
    rlim = jnp.minimum(jnp.int32(STRIPE_W), jnp.int32(N_ROWS) - r0)
    iota = _iota16()

    nfull = rlim >> 7
    rem8 = (rlim & 127) >> 3

    def cp2(c2, _):
        a0 = r0 + (c2 * 2) * 128
        a1 = a0 + 128
        pltpu.async_copy(buf.at[pl.ds(a0, 128)], cp0_v, semi0)
        pltpu.async_copy(buf.at[pl.ds(a1, 128)], cp1_v, semi1)
        pltpu.make_async_copy(buf.at[pl.ds(a0, 128)], cp0_v, semi0).wait()
        pltpu.async_copy(cp0_v, out.at[pl.ds(a0, 128)], semo0)
        pltpu.make_async_copy(buf.at[pl.ds(a1, 128)], cp1_v, semi1).wait()
        pltpu.async_copy(cp1_v, out.at[pl.ds(a1, 128)], semo1)
        pltpu.make_async_copy(cp0_v, out.at[pl.ds(a0, 128)], semo0).wait()
        pltpu.make_async_copy(cp1_v, out.at[pl.ds(a1, 128)], semo1).wait()
        return 0
    lax.fori_loop(0, nfull >> 1, cp2, 0)

    @pl.when((nfull & 1) == 1)
    def _():
        a0 = r0 + (nfull - 1) * 128
        pltpu.sync_copy(buf.at[pl.ds(a0, 128)], cp0_v)
        pltpu.sync_copy(cp0_v, out.at[pl.ds(a0, 128)])

    def cp8(k, _):
        a0 = r0 + nfull * 128 + k * 8
        pltpu.sync_copy(buf.at[pl.ds(a0, 8)], cp0_v.at[pl.ds(0, 8)])
        pltpu.sync_copy(cp0_v.at[pl.ds(0, 8)], out.at[pl.ds(a0, 8)])
        return 0
    lax.fori_loop(0, rem8, cp8, 0)

    pltpu.sync_copy(items, items_v)

    def tinit(t, _):
        table_v[pl.ds(t * 16, 16)] = jnp.full((16,), -1, I32)
        return 0
    lax.fori_loop(0, (STRIPE_W + 15) // 16, tinit, 0)

    def smax_pass(g, changed):
        ids = items_v[pl.ds(g * 16, 16)]
        bvec = iota + g * 16
        lid = ids - r0
        inm = (lid >= 0) & (lid < rlim)
        lidc = jnp.where(inm, lid, 0)
        rb = plsc.load_gather(table_v, [lidc], mask=inm)
        rb = jnp.where(inm, rb, bvec)
        fix = inm & (bvec > rb)
        plsc.store_scatter(table_v, [lidc], bvec, mask=fix)
        return changed + plsc.all_reduce_population_count(fix)

    def wcond(ch):
        return ch > 0

    def wbody(_):
        chv = lax.fori_loop(0, B // 16, smax_pass, jnp.zeros((16,), I32))
        return jnp.max(chv)

    lax.while_loop(wcond, wbody, jnp.int32(1))

    def comp(t, cntv):
        lidv = iota + t * 16
        rb = table_v[pl.ds(t * 16, 16)]
        m = (rb >= 0) & (lidv < rlim)
        mi = jnp.where(m, 1, 0).astype(I32)
        pos = cntv + plsc.cumsum(mi) - 1
        posc = jnp.where(m, pos, 0)
        plsc.store_scatter(dst_v, [posc], lidv + r0, mask=m)
        plsc.store_scatter(src_v, [posc], rb, mask=m)
        return cntv + plsc.all_reduce_population_count(m)

    cntv = lax.fori_loop(0, (STRIPE_W + 15) // 16, comp,
                         jnp.zeros((16,), I32))
    cnt = jnp.max(cntv)
    nch = (cnt + 63) >> 6

    @pl.when(cnt > 0)
    def _():
        w0d = dst_v[pl.ds(0, 16)][0]
        w0s = src_v[pl.ds(0, 16)][0]

        def padg(t, _):
            posp = iota + t * 16
            m = (posp >= cnt) & (posp < nch * 64)
            plsc.store_scatter(dst_v, [jnp.where(m, posp, 0)],
                               jnp.full((16,), 1, I32) * w0d, mask=m)
            plsc.store_scatter(src_v, [jnp.where(m, posp, 0)],
                               jnp.full((16,), 1, I32) * w0s, mask=m)
            return 0
        lax.fori_loop(cnt >> 4, jnp.minimum(nch * 4, LCAP // 16), padg, 0)

        def sc_chunk(j, _):
            for k in range(4):
                dstc_v[pl.ds(k * 16, 16)] = dst_v[pl.ds(j * 64 + k * 16, 16)]
                srcc_v[pl.ds(k * 16, 16)] = src_v[pl.ds(j * 64 + k * 16, 16)]
            pltpu.async_copy(pu.at[srcc_v], rows_v, sem).wait()
            pltpu.async_copy(rows_v, out.at[dstc_v], sem).wait()
            return 0
        lax.fori_loop(0, nch, sc_chunk, 0)


def _scatter_update(buf, items, pu):
    mesh = plsc.VectorSubcoreMesh(core_axis_name="c", subcore_axis_name="s")
    f = functools.partial(
        pl.kernel,
        out_type=[jax.ShapeDtypeStruct((N_ROWS, D), F32)],
        mesh=mesh,
        scratch_types=[
            pltpu.VMEM((B,), I32),
            pltpu.VMEM((STRIPE_W + 8,), I32),
            pltpu.VMEM((LCAP,), I32),
            pltpu.VMEM((LCAP,), I32),
            pltpu.VMEM((64,), I32),
            pltpu.VMEM((64,), I32),
            pltpu.VMEM((64, D), F32),
            pltpu.VMEM((128, D), F32),
            pltpu.VMEM((128, D), F32),
            pltpu.SemaphoreType.DMA,
            pltpu.SemaphoreType.DMA,
            pltpu.SemaphoreType.DMA,
            pltpu.SemaphoreType.DMA,
            pltpu.SemaphoreType.DMA,
        ],
        compiler_params=pltpu.CompilerParams(needs_layout_passes=False),
    )(_sc_body)
    (nb,) = f(buf, items, pu)
    return nb


BM = 512


def _prenorm(x, g, b, W1, b1, W2, b2):
    m = jnp.mean(x, axis=-1, keepdims=True)
    xc = x - m
    v = jnp.mean(xc * xc, axis=-1, keepdims=True)
    h = xc * lax.rsqrt(v + 1e-5) * g + b
    h = jnp.maximum(jnp.dot(h, W1, preferred_element_type=F32) + b1, 0.0)
    h = jnp.dot(h, W2, preferred_element_type=F32) + b2
    return h + x


def _l2n(x):
    n = jnp.sqrt(jnp.sum(x * x, axis=-1, keepdims=True))
    return x / jnp.maximum(n, 1e-12)


def _mlp_body(su, pi, pu, si, hu, hi, wcat,
              p1, p2, p3, p4, ue_o, ie_o):
    avgu = jnp.dot(hu[...], wcat[...], preferred_element_type=F32)
    avgi = jnp.dot(hi[...], wcat[...], preferred_element_type=F32)
    uf = jnp.concatenate([su[...], pi[...], avgu], axis=1)
    itf = jnp.concatenate([si[...], avgi, pu[...]], axis=1)

    def blk(p, x):
        return _prenorm(x, p['ln_g'][...], p['ln_b'][...], p['W1'][...],
                        p['b1'][...], p['W2'][...], p['b2'][...])

    ue = blk(p1, uf) + blk(p2, uf)
    ie = blk(p3, itf) + blk(p4, itf)
    ue_o[...] = _l2n(ue)
    ie_o[...] = _l2n(ie)


def _mlp(su, pi, pu, si, hu, hi, wcat, p1, p2, p3, p4):
    bspec_d = pl.BlockSpec((BM, D), lambda i: (i, 0))
    bspec_h = pl.BlockSpec((BM, HPAD), lambda i: (i, 0))
    full = lambda a: pl.BlockSpec(a.shape, lambda i: tuple(0 for _ in a.shape))
    pspec = lambda p: {k: full(v) for k, v in p.items()}
    return pl.pallas_call(
        _mlp_body,
        grid=(B // BM,),
        in_specs=[bspec_d, bspec_d, bspec_d, bspec_d, bspec_h, bspec_h,
                  full(wcat), pspec(p1), pspec(p2), pspec(p3), pspec(p4)],
        out_specs=[pl.BlockSpec((BM, 3 * D), lambda i: (i, 0))] * 2,
        out_shape=[jax.ShapeDtypeStruct((B, 3 * D), F32)] * 2,
    )(su, pi, pu, si, hu, hi, wcat, p1, p2, p3, p4)


def kernel(params, users, items, item_history_matrix, item_history_len,
           user_history_matrix, user_history_len, cates, cate_lens):
    users = users.astype(I32)
    items = items.astype(I32)
    ihm_f = item_history_matrix.astype(I32).reshape(B * HIST)
    uhm_f = user_history_matrix.astype(I32).reshape(B * HIST)
    ihl = item_history_len.astype(I32)
    uhl = user_history_len.astype(I32)
    cates_t = cates.astype(I32).reshape(N_ROWS * MC)
    clens_t = cate_lens.astype(I32)

    su, pi, pu, si, huf, hif = _gather_pool(
        params['user_emb_w'], params['item_emb_w'], cates_t, clens_t,
        users, items, ihm_f, ihl, uhm_f, uhl)

    hu = huf.reshape(B, HPAD)
    hi = hif.reshape(B, HPAD)
    wcat = jnp.pad(params['cate_emb_w'], ((0, HPAD - N_CATES), (0, 0)))

    new_buffer = _scatter_update(params['user_history_buffer'], items, pu)

    ue, ie = _mlp(su, pi, pu, si, hu, hi, wcat,
                  params['user_mlp'], params['user_mlp_2'],
                  params['item_mlp'], params['item_mlp_2'])
    return ue, ie, new_buffer
